# Initial kernel scaffold; baseline (speedup 1.0000x reference)
import jax
import jax.numpy as jnp
from jax.experimental import pallas as pl


def kernel(x, token_table, pos_table):
    raise NotImplementedError("write your pallas kernel here")


# speedup vs baseline: 3.5746x; 3.5746x over previous
import functools

import jax
import jax.numpy as jnp
from jax import lax
from jax.experimental import pallas as pl
from jax.experimental.pallas import tpu as pltpu
from jax.experimental.pallas import tpu_sc as plsc

MAXLEN = 200
EMBED_DIM = 64
BATCH = 4096
LANES = 16
NC, NS = 2, 16
NW = NC * NS
ROWS_PER_W = BATCH // NW
HALF = MAXLEN // 2
NVREG = EMBED_DIM // LANES


def _body(x_hbm, tab_hbm, pos_hbm, out_hbm, idx_v, rows_v, pos_v, sem0, sem1):
    wid = lax.axis_index("s") * NC + lax.axis_index("c")
    base = wid * ROWS_PER_W

    pltpu.sync_copy(pos_hbm, pos_v)
    sems = (sem0, sem1)

    def start_row(row, b):
        pltpu.sync_copy(x_hbm.at[row], idx_v.at[b])
        for s in range(2):
            pltpu.async_copy(
                tab_hbm.at[idx_v.at[b, s]],
                rows_v.at[b, pl.ds(s * HALF, HALF)],
                sems[b],
            )

    def finish_row(row, b):
        for s in range(2):
            pltpu.make_async_copy(
                tab_hbm.at[idx_v.at[b, s]],
                rows_v.at[b, pl.ds(s * HALF, HALF)],
                sems[b],
            ).wait()

        def add_pos(t, carry):
            for j in range(NVREG):
                plsc.addupdate(
                    rows_v.at[b, t, pl.ds(j * LANES, LANES)],
                    pos_v[t, pl.ds(j * LANES, LANES)],
                )
            return carry

        lax.fori_loop(0, MAXLEN, add_pos, 0, unroll=2)
        pltpu.sync_copy(rows_v.at[b], out_hbm.at[row])

    start_row(base, 0)

    def outer(g, carry):
        r0 = base + 2 * g
        start_row(r0 + 1, 1)
        finish_row(r0, 0)

        @pl.when(g < ROWS_PER_W // 2 - 1)
        def _():
            start_row(r0 + 2, 0)

        finish_row(r0 + 1, 1)
        return carry

    lax.fori_loop(0, ROWS_PER_W // 2, outer, 0)


_emb = pl.kernel(
    _body,
    out_type=jax.ShapeDtypeStruct((BATCH, MAXLEN, EMBED_DIM), jnp.float32),
    mesh=plsc.VectorSubcoreMesh(
        core_axis_name="c", subcore_axis_name="s", num_cores=NC, num_subcores=NS
    ),
    scratch_types=[
        pltpu.VMEM((2, 2, HALF), jnp.int32),
        pltpu.VMEM((2, MAXLEN, EMBED_DIM), jnp.float32),
        pltpu.VMEM((MAXLEN, EMBED_DIM), jnp.float32),
        pltpu.SemaphoreType.DMA,
        pltpu.SemaphoreType.DMA,
    ],
    compiler_params=pltpu.CompilerParams(use_tc_tiling_on_sc=False),
)


@jax.jit
def kernel(x, token_table, pos_table):
    x32 = x.astype(jnp.int32).reshape(BATCH, 2, HALF)
    return _emb(x32, token_table, pos_table)


# speedup vs baseline: 4.0496x; 1.1329x over previous
import jax
import jax.numpy as jnp
from jax import lax
from jax.experimental import pallas as pl
from jax.experimental.pallas import tpu as pltpu
from jax.experimental.pallas import tpu_sc as plsc

MAXLEN = 200
EMBED_DIM = 64
BATCH = 4096
LANES = 16
NC, NS = 2, 16
NW = NC * NS
ROWS_PER_W = BATCH // NW
HALF = MAXLEN // 2
NVREG = EMBED_DIM // LANES
NBUF = 4


def _body(x_hbm, tab_hbm, pos_hbm, out_hbm, idx_v, rows_v, pos_v, gsems, wsems):
    wid = lax.axis_index("s") * NC + lax.axis_index("c")
    base = wid * ROWS_PER_W

    pltpu.sync_copy(pos_hbm, pos_v)
    pltpu.sync_copy(x_hbm.at[wid], idx_v)

    def start_gather(r, b):
        for s in range(2):
            pltpu.async_copy(
                tab_hbm.at[idx_v.at[r, s]],
                rows_v.at[b, pl.ds(s * HALF, HALF)],
                gsems.at[b],
            )

    def wait_gather(b):
        for s in range(2):
            pltpu.make_async_copy(
                tab_hbm.at[idx_v.at[0, s]],
                rows_v.at[b, pl.ds(s * HALF, HALF)],
                gsems.at[b],
            ).wait()

    def add_pos(b):
        def step(t, carry):
            for j in range(NVREG):
                plsc.addupdate(
                    rows_v.at[b, t, pl.ds(j * LANES, LANES)],
                    pos_v[t, pl.ds(j * LANES, LANES)],
                )
            return carry

        lax.fori_loop(0, MAXLEN, step, 0, unroll=4)

    def wait_write(b):
        pltpu.make_async_copy(
            rows_v.at[b], out_hbm.at[base], wsems.at[b]
        ).wait()

    start_gather(0, 0)
    start_gather(1, 1)

    def outer(g, carry):
        for k in range(NBUF):
            r = g + k
            b = k
            wait_gather(b)
            add_pos(b)
            pltpu.async_copy(rows_v.at[b], out_hbm.at[base + r], wsems.at[b])
            b2 = (k + 2) % NBUF

            @pl.when(r >= 2)
            def _():
                wait_write(b2)

            @pl.when(r + 2 < ROWS_PER_W)
            def _():
                start_gather(r + 2, b2)

        return carry

    lax.fori_loop(0, ROWS_PER_W // NBUF, lambda i, c: outer(i * NBUF, c), 0)

    wait_write(2)
    wait_write(3)


_emb = pl.kernel(
    _body,
    out_type=jax.ShapeDtypeStruct((BATCH, MAXLEN, EMBED_DIM), jnp.float32),
    mesh=plsc.VectorSubcoreMesh(
        core_axis_name="c", subcore_axis_name="s", num_cores=NC, num_subcores=NS
    ),
    scratch_types=[
        pltpu.VMEM((ROWS_PER_W, 2, HALF), jnp.int32),
        pltpu.VMEM((NBUF, MAXLEN, EMBED_DIM), jnp.float32),
        pltpu.VMEM((MAXLEN, EMBED_DIM), jnp.float32),
        pltpu.SemaphoreType.DMA((NBUF,)),
        pltpu.SemaphoreType.DMA((NBUF,)),
    ],
    compiler_params=pltpu.CompilerParams(use_tc_tiling_on_sc=False),
)


@jax.jit
def kernel(x, token_table, pos_table):
    x32 = x.astype(jnp.int32).reshape(NW, ROWS_PER_W, 2, HALF)
    return _emb(x32, token_table, pos_table)


# speedup vs baseline: 4.1006x; 1.0126x over previous
import jax
import jax.numpy as jnp
from jax import lax
from jax.experimental import pallas as pl
from jax.experimental.pallas import tpu as pltpu
from jax.experimental.pallas import tpu_sc as plsc

MAXLEN = 200
MAXLEN_PAD = 256
EMBED_DIM = 64
PAD_DIM = 128
BATCH = 4096
LANES = 16
NC, NS = 2, 16
NW = NC * NS
ROWS_PER_W = BATCH // NW
SPLIT = 104
NVREG = EMBED_DIM // LANES
NBUF = 2
CHUNKS = ((0, SPLIT), (SPLIT, MAXLEN - SPLIT))


def _body(x_hbm, tab_hbm, pos_hbm, out_hbm, idx_v, rows_v, pos_v, gsems, wsems):
    wid = lax.axis_index("s") * NC + lax.axis_index("c")
    base = wid * ROWS_PER_W

    pltpu.sync_copy(pos_hbm, pos_v)
    pltpu.sync_copy(x_hbm.at[pl.ds(base, ROWS_PER_W)], idx_v)

    def start_gather(r, b):
        for off, ln in CHUNKS:
            pltpu.async_copy(
                tab_hbm.at[idx_v.at[r, pl.ds(off, ln)]],
                rows_v.at[b, pl.ds(off, ln)],
                gsems.at[b],
            )

    def wait_gather(b):
        for off, ln in CHUNKS:
            pltpu.make_async_copy(
                tab_hbm.at[idx_v.at[0, pl.ds(off, ln)]],
                rows_v.at[b, pl.ds(off, ln)],
                gsems.at[b],
            ).wait()

    def add_pos(b):
        def step(t, carry):
            for j in range(NVREG):
                plsc.addupdate(
                    rows_v.at[b, t, pl.ds(j * LANES, LANES)],
                    pos_v[t, pl.ds(j * LANES, LANES)],
                )
            return carry

        lax.fori_loop(0, MAXLEN, step, 0, unroll=4)

    def issue_write(r, b):
        pltpu.async_copy(rows_v.at[b], out_hbm.at[base + r], wsems.at[b])

    def wait_write(b):
        pltpu.make_async_copy(
            rows_v.at[b], out_hbm.at[base], wsems.at[b]
        ).wait()

    start_gather(0, 0)
    start_gather(1, 1)

    def outer(g, carry):
        for k in range(NBUF):
            r = g + k
            b = k
            wait_gather(b)
            add_pos(b)
            issue_write(r, b)

            @pl.when(r + NBUF < ROWS_PER_W)
            def _():
                wait_write(b)
                start_gather(r + NBUF, b)

        return carry

    lax.fori_loop(0, ROWS_PER_W // NBUF, lambda i, c: outer(i * NBUF, c), 0)

    for b in range(NBUF):
        wait_write(b)


_emb = pl.kernel(
    _body,
    out_type=jax.ShapeDtypeStruct((BATCH, MAXLEN, PAD_DIM), jnp.float32),
    mesh=plsc.VectorSubcoreMesh(
        core_axis_name="c", subcore_axis_name="s", num_cores=NC, num_subcores=NS
    ),
    scratch_types=[
        pltpu.VMEM((ROWS_PER_W, MAXLEN_PAD), jnp.int32),
        pltpu.VMEM((NBUF, MAXLEN, PAD_DIM), jnp.float32),
        pltpu.VMEM((MAXLEN, EMBED_DIM), jnp.float32),
        pltpu.SemaphoreType.DMA((NBUF,)),
        pltpu.SemaphoreType.DMA((NBUF,)),
    ],
    compiler_params=pltpu.CompilerParams(use_tc_tiling_on_sc=False),
)


@jax.jit
def kernel(x, token_table, pos_table):
    x32 = jnp.pad(x.astype(jnp.int32), ((0, 0), (0, MAXLEN_PAD - MAXLEN)))
    tab128 = jnp.pad(token_table, ((0, 0), (0, PAD_DIM - EMBED_DIM)))
    y = _emb(x32, tab128, pos_table)
    return y[:, :, :EMBED_DIM]


# speedup vs baseline: 7.1882x; 1.7530x over previous
import jax
import jax.numpy as jnp
from jax import lax
from jax.experimental import pallas as pl
from jax.experimental.pallas import tpu as pltpu
from jax.experimental.pallas import tpu_sc as plsc

MAXLEN = 200
MAXLEN_PAD = 256
EMBED_DIM = 64
PAD_DIM = 128
BATCH = 4096
LANES = 16
NC, NS = 2, 16
NW = NC * NS
ROWS_PER_W = BATCH // NW
SPLIT = 104
NVREG = EMBED_DIM // LANES
NBUF = 4
CHUNKS = ((0, SPLIT), (SPLIT, MAXLEN - SPLIT))


def _body(x_hbm, tab_hbm, pos_hbm, out_hbm, idx_v, rows_v, pos_v, gsems, wsems):
    wid = lax.axis_index("s") * NC + lax.axis_index("c")
    base = wid * ROWS_PER_W

    pltpu.sync_copy(pos_hbm, pos_v)
    pltpu.sync_copy(x_hbm.at[pl.ds(base, ROWS_PER_W)], idx_v)

    def start_gather(r, b):
        for off, ln in CHUNKS:
            pltpu.async_copy(
                tab_hbm.at[idx_v.at[r, pl.ds(off, ln)]],
                rows_v.at[b, pl.ds(off, ln)],
                gsems.at[b],
            )

    def wait_gather(b):
        for off, ln in CHUNKS:
            pltpu.make_async_copy(
                tab_hbm.at[idx_v.at[0, pl.ds(off, ln)]],
                rows_v.at[b, pl.ds(off, ln)],
                gsems.at[b],
            ).wait()

    def add_pos(b):
        def step(t, carry):
            for j in range(NVREG):
                plsc.addupdate(
                    rows_v.at[b, t, pl.ds(j * LANES, LANES)],
                    pos_v[t, pl.ds(j * LANES, LANES)],
                )
            return carry

        lax.fori_loop(0, MAXLEN, step, 0, unroll=4)

    def issue_write(r, b):
        pltpu.async_copy(
            rows_v.at[b],
            out_hbm.at[base + r, :, pl.ds(0, EMBED_DIM)],
            wsems.at[b],
        )

    def wait_write(b):
        pltpu.make_async_copy(
            rows_v.at[b],
            out_hbm.at[base, :, pl.ds(0, EMBED_DIM)],
            wsems.at[b],
        ).wait()

    start_gather(0, 0)
    start_gather(1, 1)

    def outer(g, carry):
        for k in range(NBUF):
            r = g + k
            b = k
            wait_gather(b)
            add_pos(b)
            issue_write(r, b)
            b2 = (k + 2) % NBUF

            @pl.when(r >= 2)
            def _():
                wait_write(b2)

            @pl.when(r + 2 < ROWS_PER_W)
            def _():
                start_gather(r + 2, b2)

        return carry

    lax.fori_loop(0, ROWS_PER_W // NBUF, lambda i, c: outer(i * NBUF, c), 0)

    wait_write(2)
    wait_write(3)


_emb = pl.kernel(
    _body,
    out_type=jax.ShapeDtypeStruct((BATCH, MAXLEN, PAD_DIM), jnp.float32),
    mesh=plsc.VectorSubcoreMesh(
        core_axis_name="c", subcore_axis_name="s", num_cores=NC, num_subcores=NS
    ),
    scratch_types=[
        pltpu.VMEM((ROWS_PER_W, MAXLEN_PAD), jnp.int32),
        pltpu.VMEM((NBUF, MAXLEN, EMBED_DIM), jnp.float32),
        pltpu.VMEM((MAXLEN, EMBED_DIM), jnp.float32),
        pltpu.SemaphoreType.DMA((NBUF,)),
        pltpu.SemaphoreType.DMA((NBUF,)),
    ],
    compiler_params=pltpu.CompilerParams(use_tc_tiling_on_sc=False),
)


@jax.jit
def kernel(x, token_table, pos_table):
    x32 = jnp.pad(x.astype(jnp.int32), ((0, 0), (0, MAXLEN_PAD - MAXLEN)))
    y = _emb(x32, token_table, pos_table)
    return y[:, :, :EMBED_DIM]
